# Initial kernel scaffold; baseline (speedup 1.0000x reference)
import jax
import jax.numpy as jnp
from jax.experimental import pallas as pl


def kernel(x, tables):
    raise NotImplementedError("write your pallas kernel here")


# speedup vs baseline: 64.8812x; 64.8812x over previous
import functools
import math

import jax
import jax.numpy as jnp
import numpy as np
from jax import lax
from jax.experimental import pallas as pl
from jax.experimental.pallas import tpu as pltpu
from jax.experimental.pallas import tpu_sc as plsc

_N_LEVELS = 16
_BASE_RES = 16
_FINEST_RES = 512
_LOG2_HASH = 19
_B = math.exp((math.log(_FINEST_RES) - math.log(_BASE_RES)) / (_N_LEVELS - 1))
_P1 = 2654435761
_P2 = 805459861
_P1C = np.int32(np.uint32(_P1 & 0xFFFFFFFF))
_P2C = np.int32(np.uint32(_P2 & 0xFFFFFFFF))
_PH1, _PL1 = _P1 >> 16, _P1 & 0xFFFF
_PH2, _PL2 = _P2 >> 16, _P2 & 0xFFFF


def _level_specs():
    specs = []
    for i in range(_N_LEVELS):
        res = math.floor(_BASE_RES * _B**i)
        n_dense = res**3
        num_entries = n_dense if n_dense < 2**_LOG2_HASH else 2**_LOG2_HASH
        specs.append((res, num_entries))
    return specs


_SPECS = _level_specs()
_OFFS = [(dx, dy, dz) for dx in (0, 1) for dy in (0, 1) for dz in (0, 1)]

_NW = 32
_N_PTS = 4 * 65536
_PTS_PER_W = _N_PTS // _NW
_C = 1024
_G = _C // 16
_NCH = _PTS_PER_W // _C
_NIDX = 16 * _C
_D = 512
_ND = _NIDX // _D


def _make_kernel():
    mesh = plsc.VectorSubcoreMesh(core_axis_name="c", subcore_axis_name="s")

    @functools.partial(
        pl.kernel,
        out_type=jax.ShapeDtypeStruct((32, _N_PTS), jnp.float32),
        mesh=mesh,
        compiler_params=pltpu.CompilerParams(use_tc_tiling_on_sc=False),
        scratch_types=[
            pltpu.VMEM((_C,), jnp.float32),
            pltpu.VMEM((_C,), jnp.float32),
            pltpu.VMEM((_C,), jnp.float32),
            pltpu.VMEM((_NIDX,), jnp.int32),
            pltpu.VMEM((_NIDX,), jnp.float32),
            pltpu.VMEM((32, _C), jnp.float32),
            pltpu.SemaphoreType.DMA,
        ],
    )
    def embed(xs_h, ys_h, zs_h, *rest):
        tabs = rest[:_N_LEVELS]
        out_h = rest[_N_LEVELS]
        xv, yv, zv, idxb, rows, outc, sem = rest[_N_LEVELS + 1:]

        wid = lax.axis_index("s") * jnp.int32(2) + lax.axis_index("c")
        base = wid * jnp.int32(_PTS_PER_W)
        iota = lax.iota(jnp.int32, 16)
        iota32 = iota * jnp.int32(32)

        def coords(g, resf):
            s = pl.ds(g * jnp.int32(16), 16)
            xs = xv[s] * resf
            ys = yv[s] * resf
            zs = zv[s] * resf
            xi = xs.astype(jnp.int32)
            yi = ys.astype(jnp.int32)
            zi = zs.astype(jnp.int32)
            return xs, ys, zs, xi, yi, zi

        def make_pass_a(res, n):
            resf = jnp.float32(res)
            pow2 = (n & (n - 1)) == 0
            nn = jnp.int32(n)

            def body(g, carry):
                _, _, _, xi, yi, zi = coords(g, resf)
                gb = g * jnp.int32(128)
                xi1 = xi + jnp.int32(1)
                ly0 = yi * _P1C
                ly1 = ly0 + _P1C
                lz0 = zi * _P2C
                lz1 = lz0 + _P2C
                if pow2:
                    mask = jnp.int32(n - 1)
                    for ci, (dx, dy, dz) in enumerate(_OFFS):
                        v = ((xi1 if dx else xi)
                             ^ (ly1 if dy else ly0)
                             ^ (lz1 if dz else lz0))
                        v = v & mask
                        off = gb + jnp.int32(ci * 16)
                        idxb[pl.ds(off, 16)] = v
                        idxb[pl.ds(off + jnp.int32(_NIDX // 2), 16)] = v + nn
                else:
                    yi1 = yi + jnp.int32(1)
                    zi1 = zi + jnp.int32(1)
                    hy0 = (yi * jnp.int32(_PH1)
                           + ((yi * jnp.int32(_PL1)) >> 16)) >> 16
                    hy1 = (yi1 * jnp.int32(_PH1)
                           + ((yi1 * jnp.int32(_PL1)) >> 16)) >> 16
                    hz0 = (zi * jnp.int32(_PH2)
                           + ((zi * jnp.int32(_PL2)) >> 16)) >> 16
                    hz1 = (zi1 * jnp.int32(_PH2)
                           + ((zi1 * jnp.int32(_PL2)) >> 16)) >> 16
                    c1 = jnp.int32((1 << 32) % n)
                    c2 = jnp.int32((1 << 31) % n)
                    lowmask = jnp.int32(0x7FFFFFFF)
                    for ci, (dx, dy, dz) in enumerate(_OFFS):
                        vlo = ((xi1 if dx else xi)
                               ^ (ly1 if dy else ly0)
                               ^ (lz1 if dz else lz0))
                        vhi = (hy1 if dy else hy0) ^ (hz1 if dz else hz0)
                        b31 = lax.shift_right_logical(vlo, jnp.int32(31))
                        r = vlo & lowmask
                        s_val = vhi * c1 + b31 * c2 + lax.rem(r, nn)
                        v = lax.rem(s_val, nn)
                        off = gb + jnp.int32(ci * 16)
                        idxb[pl.ds(off, 16)] = v
                        idxb[pl.ds(off + jnp.int32(_NIDX // 2), 16)] = v + nn
                return carry

            return body

        def make_pass_b(lvl, res):
            resf = jnp.float32(res)
            col = 2 * lvl

            def body(g, carry):
                xs, ys, zs, xi, yi, zi = coords(g, resf)
                wx = xs - xi.astype(jnp.float32)
                wy = ys - yi.astype(jnp.float32)
                wz = zs - zi.astype(jnp.float32)
                omx = jnp.float32(1.0) - wx
                omy = jnp.float32(1.0) - wy
                omz = jnp.float32(1.0) - wz
                gb = g * jnp.int32(128)
                po = g * jnp.int32(16)
                for ff in range(2):
                    fb = gb + jnp.int32(ff * (_NIDX // 2))
                    f = [rows[pl.ds(fb + jnp.int32(ci * 16), 16)]
                         for ci in range(8)]
                    c00 = f[0] * omx + f[4] * wx
                    c01 = f[1] * omx + f[5] * wx
                    c10 = f[2] * omx + f[6] * wx
                    c11 = f[3] * omx + f[7] * wx
                    c0 = c00 * omy + c10 * wy
                    c1 = c01 * omy + c11 * wy
                    cc = c0 * omz + c1 * wz
                    outc[col + ff, pl.ds(po, 16)] = cc
                return carry

            return body

        def chunk_body(ch, carry):
            pbase = base + ch * jnp.int32(_C)
            pltpu.sync_copy(xs_h.at[pl.ds(pbase, _C)], xv)
            pltpu.sync_copy(ys_h.at[pl.ds(pbase, _C)], yv)
            pltpu.sync_copy(zs_h.at[pl.ds(pbase, _C)], zv)
            for lvl in range(_N_LEVELS):
                res, n = _SPECS[lvl]
                tab = tabs[lvl]
                lax.fori_loop(jnp.int32(0), jnp.int32(_G),
                              make_pass_a(res, n), 0)

                def fire(j, c2_, tab=tab):
                    s = pl.ds(j * jnp.int32(_D), _D)
                    pltpu.make_async_copy(
                        tab.at[idxb.at[s]], rows.at[s], sem).start()
                    return c2_

                def drain(j, c2_, tab=tab):
                    s = pl.ds(j * jnp.int32(_D), _D)
                    pltpu.make_async_copy(
                        tab.at[idxb.at[s]], rows.at[s], sem).wait()
                    return c2_

                lax.fori_loop(jnp.int32(0), jnp.int32(_ND), fire, 0)
                lax.fori_loop(jnp.int32(0), jnp.int32(_ND), drain, 0)
                lax.fori_loop(jnp.int32(0), jnp.int32(_G),
                              make_pass_b(lvl, res), 0)
            pltpu.sync_copy(outc, out_h.at[:, pl.ds(pbase, _C)])
            return carry

        lax.fori_loop(jnp.int32(0), jnp.int32(_NCH), chunk_body, 0)

    return embed


_EMBED = _make_kernel()


@jax.jit
def kernel(x, tables):
    batch, num_points, _ = x.shape
    xf = x.reshape(-1, 3).astype(jnp.float32)
    xs = xf[:, 0]
    ys = xf[:, 1]
    zs = xf[:, 2]
    tabs = [jnp.concatenate([t[:, 0], t[:, 1]]) for t in tables]
    out = _EMBED(xs, ys, zs, *tabs)
    return out.T.reshape(batch, num_points, 32)


# speedup vs baseline: 91.4014x; 1.4088x over previous
import functools
import math

import jax
import jax.numpy as jnp
import numpy as np
from jax import lax
from jax.experimental import pallas as pl
from jax.experimental.pallas import tpu as pltpu
from jax.experimental.pallas import tpu_sc as plsc

_N_LEVELS = 16
_BASE_RES = 16
_FINEST_RES = 512
_LOG2_HASH = 19
_B = math.exp((math.log(_FINEST_RES) - math.log(_BASE_RES)) / (_N_LEVELS - 1))
_P1 = 2654435761
_P2 = 805459861
_P1C = np.int32(np.uint32(_P1 & 0xFFFFFFFF))
_P2C = np.int32(np.uint32(_P2 & 0xFFFFFFFF))
_PH1, _PL1 = _P1 >> 16, _P1 & 0xFFFF
_PH2, _PL2 = _P2 >> 16, _P2 & 0xFFFF


def _level_specs():
    specs = []
    for i in range(_N_LEVELS):
        res = math.floor(_BASE_RES * _B**i)
        n_dense = res**3
        num_entries = n_dense if n_dense < 2**_LOG2_HASH else 2**_LOG2_HASH
        specs.append((res, num_entries))
    return specs


_SPECS = _level_specs()
_OFFS = [(dx, dy, dz) for dx in (0, 1) for dy in (0, 1) for dz in (0, 1)]
_SPLIT_K = {8000: 16, 15625: 16, 64000: 16, 125000: 17, 512000: 19}

_NW = 32
_N_PTS = 4 * 65536
_PTS_PER_W = _N_PTS // _NW
_C = 1024
_G = _C // 16
_NCH = _PTS_PER_W // _C
_NIDX = 16 * _C
_D = 512
_ND = _NIDX // _D


def _make_kernel():
    mesh = plsc.VectorSubcoreMesh(core_axis_name="c", subcore_axis_name="s")

    @functools.partial(
        pl.kernel,
        out_type=jax.ShapeDtypeStruct((32, _N_PTS), jnp.float32),
        mesh=mesh,
        compiler_params=pltpu.CompilerParams(use_tc_tiling_on_sc=False),
        scratch_types=[
            pltpu.VMEM((_C,), jnp.float32),
            pltpu.VMEM((_C,), jnp.float32),
            pltpu.VMEM((_C,), jnp.float32),
            pltpu.VMEM((_NIDX,), jnp.int32),
            pltpu.VMEM((_NIDX,), jnp.int32),
            pltpu.VMEM((_NIDX,), jnp.float32),
            pltpu.VMEM((_NIDX,), jnp.float32),
            pltpu.VMEM((32, _C), jnp.float32),
            pltpu.SemaphoreType.DMA,
            pltpu.SemaphoreType.DMA,
        ],
    )
    def embed(xs_h, ys_h, zs_h, *rest):
        tabs = rest[:_N_LEVELS]
        out_h = rest[_N_LEVELS]
        (xv, yv, zv, idx0, idx1, rows0, rows1, outc,
         sem0, sem1) = rest[_N_LEVELS + 1:]
        idxbs = (idx0, idx1)
        rowss = (rows0, rows1)
        sems = (sem0, sem1)

        wid = lax.axis_index("s") * jnp.int32(2) + lax.axis_index("c")
        base = wid * jnp.int32(_PTS_PER_W)
        iota = lax.iota(jnp.int32, 16)
        iota32 = iota * jnp.int32(32)

        def coords(g, resf):
            s = pl.ds(g * jnp.int32(16), 16)
            xs = xv[s] * resf
            ys = yv[s] * resf
            zs = zv[s] * resf
            xi = xs.astype(jnp.int32)
            yi = ys.astype(jnp.int32)
            zi = zs.astype(jnp.int32)
            return xs, ys, zs, xi, yi, zi

        def make_pass_a(res, n, idxb):
            resf = jnp.float32(res)
            pow2 = (n & (n - 1)) == 0
            nn = jnp.int32(n)

            def body(g, carry):
                _, _, _, xi, yi, zi = coords(g, resf)
                gb = g * jnp.int32(128)
                xi1 = xi + jnp.int32(1)
                ly0 = yi * _P1C
                ly1 = ly0 + _P1C
                lz0 = zi * _P2C
                lz1 = lz0 + _P2C
                if pow2:
                    mask = jnp.int32(n - 1)
                    for ci, (dx, dy, dz) in enumerate(_OFFS):
                        v = ((xi1 if dx else xi)
                             ^ (ly1 if dy else ly0)
                             ^ (lz1 if dz else lz0))
                        v = v & mask
                        off = gb + jnp.int32(ci * 16)
                        idxb[pl.ds(off, 16)] = v
                        idxb[pl.ds(off + jnp.int32(_NIDX // 2), 16)] = v + nn
                else:
                    yi1 = yi + jnp.int32(1)
                    zi1 = zi + jnp.int32(1)
                    hy0 = (yi * jnp.int32(_PH1)
                           + ((yi * jnp.int32(_PL1)) >> 16)) >> 16
                    hy1 = (yi1 * jnp.int32(_PH1)
                           + ((yi1 * jnp.int32(_PL1)) >> 16)) >> 16
                    hz0 = (zi * jnp.int32(_PH2)
                           + ((zi * jnp.int32(_PL2)) >> 16)) >> 16
                    hz1 = (zi1 * jnp.int32(_PH2)
                           + ((zi1 * jnp.int32(_PL2)) >> 16)) >> 16
                    k = _SPLIT_K[n]
                    c1 = jnp.int32((1 << 32) % n)
                    c3 = jnp.int32((1 << k) % n)
                    kmask = jnp.int32((1 << k) - 1)
                    invn = jnp.float32(np.float32(1.0) / np.float32(n))
                    nf = jnp.int32(n)
                    h00 = (hy0 ^ hz0) * c1
                    h01 = (hy0 ^ hz1) * c1
                    h10 = (hy1 ^ hz0) * c1
                    h11 = (hy1 ^ hz1) * c1
                    hcs = ((h00, h01), (h10, h11))
                    for ci, (dx, dy, dz) in enumerate(_OFFS):
                        vlo = ((xi1 if dx else xi)
                               ^ (ly1 if dy else ly0)
                               ^ (lz1 if dz else lz0))
                        rh = lax.shift_right_logical(vlo, jnp.int32(k))
                        rl = vlo & kmask
                        s_val = hcs[dy][dz] + rh * c3 + rl
                        q = (s_val.astype(jnp.float32) * invn).astype(jnp.int32)
                        v = s_val - q * nf
                        v = jnp.where(v < 0, v + nf, v)
                        v = jnp.where(v >= nf, v - nf, v)
                        off = gb + jnp.int32(ci * 16)
                        idxb[pl.ds(off, 16)] = v
                        idxb[pl.ds(off + jnp.int32(_NIDX // 2), 16)] = v + nn
                return carry

            return body

        def make_pass_b(lvl, res, rows):
            resf = jnp.float32(res)
            col = 2 * lvl

            def body(g, carry):
                xs, ys, zs, xi, yi, zi = coords(g, resf)
                wx = xs - xi.astype(jnp.float32)
                wy = ys - yi.astype(jnp.float32)
                wz = zs - zi.astype(jnp.float32)
                omx = jnp.float32(1.0) - wx
                omy = jnp.float32(1.0) - wy
                omz = jnp.float32(1.0) - wz
                gb = g * jnp.int32(128)
                po = g * jnp.int32(16)
                for ff in range(2):
                    fb = gb + jnp.int32(ff * (_NIDX // 2))
                    f = [rows[pl.ds(fb + jnp.int32(ci * 16), 16)]
                         for ci in range(8)]
                    c00 = f[0] * omx + f[4] * wx
                    c01 = f[1] * omx + f[5] * wx
                    c10 = f[2] * omx + f[6] * wx
                    c11 = f[3] * omx + f[7] * wx
                    c0 = c00 * omy + c10 * wy
                    c1 = c01 * omy + c11 * wy
                    cc = c0 * omz + c1 * wz
                    outc[col + ff, pl.ds(po, 16)] = cc
                return carry

            return body

        def gen_and_fire(lvl):
            b = lvl % 2
            res, n = _SPECS[lvl]
            tab = tabs[lvl]
            idxb, rows, sem = idxbs[b], rowss[b], sems[b]
            lax.fori_loop(jnp.int32(0), jnp.int32(_G),
                          make_pass_a(res, n, idxb), 0)

            def fire(j, c2_):
                s = pl.ds(j * jnp.int32(_D), _D)
                pltpu.make_async_copy(
                    tab.at[idxb.at[s]], rows.at[s], sem).start()
                return c2_

            lax.fori_loop(jnp.int32(0), jnp.int32(_ND), fire, 0)

        def drain_and_combine(lvl):
            b = lvl % 2
            res, n = _SPECS[lvl]
            tab = tabs[lvl]
            idxb, rows, sem = idxbs[b], rowss[b], sems[b]

            def drain(j, c2_):
                s = pl.ds(j * jnp.int32(_D), _D)
                pltpu.make_async_copy(
                    tab.at[idxb.at[s]], rows.at[s], sem).wait()
                return c2_

            lax.fori_loop(jnp.int32(0), jnp.int32(_ND), drain, 0)
            lax.fori_loop(jnp.int32(0), jnp.int32(_G),
                          make_pass_b(lvl, res, rows), 0)

        def chunk_body(ch, carry):
            pbase = base + ch * jnp.int32(_C)
            pltpu.sync_copy(xs_h.at[pl.ds(pbase, _C)], xv)
            pltpu.sync_copy(ys_h.at[pl.ds(pbase, _C)], yv)
            pltpu.sync_copy(zs_h.at[pl.ds(pbase, _C)], zv)
            gen_and_fire(0)
            for lvl in range(_N_LEVELS):
                if lvl + 1 < _N_LEVELS:
                    gen_and_fire(lvl + 1)
                drain_and_combine(lvl)
            pltpu.sync_copy(outc, out_h.at[:, pl.ds(pbase, _C)])
            return carry

        lax.fori_loop(jnp.int32(0), jnp.int32(_NCH), chunk_body, 0)

    return embed


_EMBED = _make_kernel()


@jax.jit
def kernel(x, tables):
    batch, num_points, _ = x.shape
    xf = x.reshape(-1, 3).astype(jnp.float32)
    xs = xf[:, 0]
    ys = xf[:, 1]
    zs = xf[:, 2]
    tabs = [jnp.concatenate([t[:, 0], t[:, 1]]) for t in tables]
    out = _EMBED(xs, ys, zs, *tabs)
    return out.T.reshape(batch, num_points, 32)
